# Initial kernel scaffold; baseline (speedup 1.0000x reference)
import jax
import jax.numpy as jnp
from jax.experimental import pallas as pl


def kernel(x, edge_index, W1_src, W1_dst, att1_src, att1_dst, b1, W2_src, W2_dst, att2_src, att2_dst, b2, Wf1, bf1, gamma, beta, Wf2, bf2):
    raise NotImplementedError("write your pallas kernel here")


# speedup vs baseline: 14.7616x; 14.7616x over previous
import functools

import jax
import jax.numpy as jnp
from jax import lax
from jax.experimental import pallas as pl
from jax.experimental.pallas import tpu as pltpu
from jax.experimental.pallas import tpu_sc as plsc

N = 10000
D = 128
E = 320000
HID = 128
OUT = 64
NEG = 0.2

NP = 10240
SCW = 80
WA = 80
WB = 48
DEN = 48
GB = 1024
GRID = NP // GB

NC, NS, L = 2, 16, 16
EPT = E // NS
CH = 80
NCHUNK = EPT // CH
RPT = NP // NS
ZR = 128


def _project_tail(i, feat_list, ws_ref, wd_ref, ats_ref, atd_ref,
                  aug_ref, as_ref, ad_ref, ms_ref, md_ref):
    xs = sum(jnp.dot(f, ws_ref[pl.ds(o, f.shape[1]), :],
                     preferred_element_type=jnp.float32)
             for f, o in feat_list)
    xd = sum(jnp.dot(f, wd_ref[pl.ds(o, f.shape[1]), :],
                     preferred_element_type=jnp.float32)
             for f, o in feat_list)
    avs = jnp.sum(xs * ats_ref[...], axis=1, keepdims=True)
    avd = jnp.sum(xd * atd_ref[...], axis=1, keepdims=True)
    ones = jnp.ones((GB, 1), jnp.float32)
    zpad = jnp.zeros((GB, SCW - WB - 1), jnp.float32)
    aug_ref[0] = xs[:, :WA]
    aug_ref[1] = jnp.concatenate([xs[:, WA:D], ones, zpad], axis=1)
    as_ref[...] = avs
    ad_ref[...] = avd

    @pl.when(i == 0)
    def _():
        ms_ref[...] = jnp.full((1, 1), -jnp.inf, jnp.float32)
        md_ref[...] = jnp.full((1, 1), -jnp.inf, jnp.float32)

    ms_ref[...] = jnp.maximum(ms_ref[...], jnp.broadcast_to(jnp.max(avs), (1, 1)))
    md_ref[...] = jnp.maximum(md_ref[...], jnp.broadcast_to(jnp.max(avd), (1, 1)))


def _normalize(part_ref, augp_ref, asp_ref, adp_ref, msp_ref, mdp_ref,
               ba_ref, bb_ref):
    p0 = part_ref[0]
    p1 = part_ref[1]
    sel = jnp.where(
        lax.broadcasted_iota(jnp.int32, (SCW, 1), 0) == DEN, 1.0, 0.0
    ).astype(jnp.float32)
    den = jnp.dot(p1, sel, preferred_element_type=jnp.float32)
    m11 = jnp.maximum(msp_ref[...] + mdp_ref[...], 0.0)
    z = asp_ref[...] + adp_ref[...]
    alpha = jnp.where(z > 0, z, NEG * z)
    wself = jnp.exp(alpha - m11)
    numa = p0 + wself * augp_ref[0]
    numb = p1[:, :WB] + wself * augp_ref[1][:, :WB]
    dent = den + wself + 1e-16
    ha = jnp.maximum(numa / dent + ba_ref[...], 0.0)
    hb = jnp.maximum(numb / dent + bb_ref[...], 0.0)
    return ha, hb


_NORM_SPECS = [
    pl.BlockSpec((NC, GB, SCW), lambda i: (0, i, 0)),
    pl.BlockSpec((NC, GB, SCW), lambda i: (0, i, 0)),
    pl.BlockSpec((GB, 1), lambda i: (i, 0)),
    pl.BlockSpec((GB, 1), lambda i: (i, 0)),
    pl.BlockSpec((1, 1), lambda i: (0, 0)),
    pl.BlockSpec((1, 1), lambda i: (0, 0)),
    pl.BlockSpec((1, WA), lambda i: (0, 0)),
    pl.BlockSpec((1, WB), lambda i: (0, 0)),
]

_PROJ_IN_SPECS = [
    pl.BlockSpec((D, HID), lambda i: (0, 0)),
    pl.BlockSpec((D, HID), lambda i: (0, 0)),
    pl.BlockSpec((1, HID), lambda i: (0, 0)),
    pl.BlockSpec((1, HID), lambda i: (0, 0)),
]

_PROJ_OUT_SPECS = [
    pl.BlockSpec((NC, GB, SCW), lambda i: (0, i, 0)),
    pl.BlockSpec((GB, 1), lambda i: (i, 0)),
    pl.BlockSpec((GB, 1), lambda i: (i, 0)),
    pl.BlockSpec((1, 1), lambda i: (0, 0)),
    pl.BlockSpec((1, 1), lambda i: (0, 0)),
]

_PROJ_OUT_SHAPE = [
    jax.ShapeDtypeStruct((NC, NP, SCW), jnp.float32),
    jax.ShapeDtypeStruct((NP, 1), jnp.float32),
    jax.ShapeDtypeStruct((NP, 1), jnp.float32),
    jax.ShapeDtypeStruct((1, 1), jnp.float32),
    jax.ShapeDtypeStruct((1, 1), jnp.float32),
]


def _project(feat, ws, wd, ats, atd):
    def body(f_ref, ws_ref, wd_ref, ats_ref, atd_ref,
             aug_ref, as_ref, ad_ref, ms_ref, md_ref):
        i = pl.program_id(0)
        _project_tail(i, [(f_ref[...], 0)], ws_ref, wd_ref, ats_ref, atd_ref,
                      aug_ref, as_ref, ad_ref, ms_ref, md_ref)

    return pl.pallas_call(
        body,
        grid=(GRID,),
        in_specs=[pl.BlockSpec((GB, D), lambda i: (i, 0))] + _PROJ_IN_SPECS,
        out_specs=_PROJ_OUT_SPECS,
        out_shape=_PROJ_OUT_SHAPE,
    )(feat, ws, wd, ats, atd)


def _norm_project(part, augp, asp, adp, msp, mdp, ba, bb, ws, wd, ats, atd):
    def body(part_ref, augp_ref, asp_ref, adp_ref, msp_ref, mdp_ref,
             ba_ref, bb_ref, ws_ref, wd_ref, ats_ref, atd_ref,
             aug_ref, as_ref, ad_ref, ms_ref, md_ref):
        i = pl.program_id(0)
        ha, hb = _normalize(part_ref, augp_ref, asp_ref, adp_ref, msp_ref,
                            mdp_ref, ba_ref, bb_ref)
        _project_tail(i, [(ha, 0), (hb, WA)], ws_ref, wd_ref, ats_ref,
                      atd_ref, aug_ref, as_ref, ad_ref, ms_ref, md_ref)

    return pl.pallas_call(
        body,
        grid=(GRID,),
        in_specs=_NORM_SPECS + _PROJ_IN_SPECS,
        out_specs=_PROJ_OUT_SPECS,
        out_shape=_PROJ_OUT_SHAPE,
    )(part, augp, asp, adp, msp, mdp, ba, bb, ws, wd, ats, atd)


def _head(part, augp, asp, adp, msp, mdp, ba, bb,
          wf1, bf1, gamma, beta, wf2, bf2):
    bn_scale = float(1.0 / (1.0 + 1e-5) ** 0.5)

    def body(part_ref, augp_ref, asp_ref, adp_ref, msp_ref, mdp_ref,
             ba_ref, bb_ref, wf1_ref, bf1_ref, g_ref, be_ref, wf2_ref,
             bf2_ref, y_ref):
        ha, hb = _normalize(part_ref, augp_ref, asp_ref, adp_ref, msp_ref,
                            mdp_ref, ba_ref, bb_ref)
        f = (jnp.dot(ha, wf1_ref[pl.ds(0, WA), :],
                     preferred_element_type=jnp.float32)
             + jnp.dot(hb, wf1_ref[pl.ds(WA, WB), :],
                       preferred_element_type=jnp.float32))
        f = f + bf1_ref[...]
        f = g_ref[...] * (f * bn_scale) + be_ref[...]
        f = jnp.maximum(f, 0.0)
        g = jnp.dot(f, wf2_ref[...], preferred_element_type=jnp.float32)
        g = g + bf2_ref[...]
        mx = jnp.max(g, axis=1, keepdims=True)
        eg = jnp.exp(g - mx)
        y_ref[...] = g - mx - jnp.log(jnp.sum(eg, axis=1, keepdims=True))

    return pl.pallas_call(
        body,
        grid=(GRID,),
        in_specs=_NORM_SPECS + [
            pl.BlockSpec((HID, OUT), lambda i: (0, 0)),
            pl.BlockSpec((1, OUT), lambda i: (0, 0)),
            pl.BlockSpec((1, OUT), lambda i: (0, 0)),
            pl.BlockSpec((1, OUT), lambda i: (0, 0)),
            pl.BlockSpec((OUT, OUT), lambda i: (0, 0)),
            pl.BlockSpec((1, OUT), lambda i: (0, 0)),
        ],
        out_specs=pl.BlockSpec((GB, OUT), lambda i: (i, 0)),
        out_shape=jax.ShapeDtypeStruct((NP, OUT), jnp.float32),
    )(part, augp, asp, adp, msp, mdp, ba, bb,
      wf1, bf1, gamma, beta, wf2, bf2)


def _edge_pass(src, dst, asrc, adst, mvec, xstk):
    mesh = plsc.VectorSubcoreMesh(core_axis_name="c", subcore_axis_name="s")

    @functools.partial(
        pl.kernel,
        out_type=jax.ShapeDtypeStruct((NC, NP, SCW), jnp.float32),
        mesh=mesh,
        compiler_params=pltpu.CompilerParams(
            needs_layout_passes=False, use_tc_tiling_on_sc=False),
        scratch_types=[
            pltpu.VMEM((NP,), jnp.float32),
            pltpu.VMEM((NP,), jnp.float32),
            pltpu.VMEM((L,), jnp.float32),
            pltpu.VMEM((CH,), jnp.int32),
            pltpu.VMEM((CH,), jnp.int32),
            pltpu.VMEM((CH,), jnp.float32),
            pltpu.VMEM((CH, SCW), jnp.float32),
            pltpu.VMEM((ZR, SCW), jnp.float32),
            pltpu.VMEM_SHARED((NP, SCW), jnp.float32),
            pltpu.SemaphoreType.DMA,
        ],
    )
    def k(src_h, dst_h, as_h, ad_h, m_h, xs_h, out_h,
          as_v, ad_v, m_v, si_v, di_v, w_v, rows_v, z_v, acc_sh, sem):
        c = lax.axis_index("c")
        s = lax.axis_index("s")

        pltpu.sync_copy(as_h, as_v)
        pltpu.sync_copy(ad_h, ad_v)
        pltpu.sync_copy(m_h, m_v)

        zeros16 = jnp.zeros((L,), jnp.float32)

        def zfill(r, carry):
            for j in range(SCW // L):
                z_v[r, pl.ds(j * L, L)] = zeros16
            return carry

        lax.fori_loop(0, ZR, zfill, 0)

        row0 = s * RPT

        def zcopy(kk, carry):
            pltpu.sync_copy(z_v, acc_sh.at[pl.ds(row0 + kk * ZR, ZR)])
            return carry

        lax.fori_loop(0, RPT // ZR, zcopy, 0)

        m16 = m_v[...]
        coff = jnp.full((L,), c * NP, jnp.int32)
        plsc.subcore_barrier()

        ebase = s * EPT

        def chunk(kk, carry):
            base = ebase + kk * CH
            pltpu.sync_copy(src_h.at[pl.ds(base, CH)], si_v)
            pltpu.sync_copy(dst_h.at[pl.ds(base, CH)], di_v)

            def wbody(i, carry2):
                si = si_v[pl.ds(i * L, L)]
                di = di_v[pl.ds(i * L, L)]
                av = plsc.load_gather(as_v, [si])
                bv = plsc.load_gather(ad_v, [di])
                zz = av + bv
                al = jnp.where(zz > 0, zz, NEG * zz)
                w_v[pl.ds(i * L, L)] = jnp.exp(al - m16)
                si_v[pl.ds(i * L, L)] = si + coff
                return carry2

            lax.fori_loop(0, CH // L, wbody, 0)

            pltpu.async_copy(xs_h.at[si_v], rows_v, sem).wait()

            def mbody(r, carry2):
                wb = plsc.load_gather(w_v, [jnp.full((L,), r, jnp.int32)])
                for j in range(SCW // L):
                    rows_v[r, pl.ds(j * L, L)] = rows_v[r, pl.ds(j * L, L)] * wb
                return carry2

            lax.fori_loop(0, CH, mbody, 0)

            pltpu.sync_copy(rows_v, acc_sh.at[di_v], add=True)
            return carry

        lax.fori_loop(0, NCHUNK, chunk, 0)
        plsc.subcore_barrier()

        def wback(kk, carry):
            pltpu.sync_copy(acc_sh.at[pl.ds(row0 + kk * ZR, ZR)],
                            out_h.at[c, pl.ds(row0 + kk * ZR, ZR)])
            return carry

        lax.fori_loop(0, RPT // ZR, wback, 0)

    return k(src, dst, asrc, adst, mvec, xstk)


def kernel(x, edge_index, W1_src, W1_dst, att1_src, att1_dst, b1,
           W2_src, W2_dst, att2_src, att2_dst, b2,
           Wf1, bf1, gamma, beta, Wf2, bf2):
    xp = jnp.pad(x, ((0, NP - N), (0, 0)))
    src = edge_index[0].astype(jnp.int32)
    dst = edge_index[1].astype(jnp.int32)

    aug1, as1, ad1, ms1, md1 = _project(
        xp, W1_src, W1_dst, att1_src.reshape(1, -1), att1_dst.reshape(1, -1))
    m1 = jnp.maximum(ms1[0, 0] + md1[0, 0], 0.0)
    part1 = _edge_pass(src, dst, as1.reshape(NP), ad1.reshape(NP),
                       jnp.full((L,), 1.0, jnp.float32) * m1,
                       aug1.reshape(NC * NP, SCW))

    b1a, b1b = b1[:WA].reshape(1, -1), b1[WA:].reshape(1, -1)
    aug2, as2, ad2, ms2, md2 = _norm_project(
        part1, aug1, as1, ad1, ms1, md1, b1a, b1b,
        W2_src, W2_dst, att2_src.reshape(1, -1), att2_dst.reshape(1, -1))
    m2 = jnp.maximum(ms2[0, 0] + md2[0, 0], 0.0)
    part2 = _edge_pass(src, dst, as2.reshape(NP), ad2.reshape(NP),
                       jnp.full((L,), 1.0, jnp.float32) * m2,
                       aug2.reshape(NC * NP, SCW))

    b2a, b2b = b2[:WA].reshape(1, -1), b2[WA:].reshape(1, -1)
    y = _head(part2, aug2, as2, ad2, ms2, md2, b2a, b2b,
              Wf1, bf1.reshape(1, -1), gamma.reshape(1, -1),
              beta.reshape(1, -1), Wf2, bf2.reshape(1, -1))
    return y[:N]


# speedup vs baseline: 17.6353x; 1.1947x over previous
import functools

import jax
import jax.numpy as jnp
from jax import lax
from jax.experimental import pallas as pl
from jax.experimental.pallas import tpu as pltpu
from jax.experimental.pallas import tpu_sc as plsc

N = 10000
D = 128
E = 320000
HID = 128
OUT = 64
NEG = 0.2

NP = 10240
HW = 64
SCW = 80
DEN = 64
GB = 1024
GRID = NP // GB

NC, NS, L = 2, 16, 16
CH = 80
NCHUNK = 256
EPT = CH * NCHUNK
EP = NS * EPT
RPT = NP // NS
ZR = 64


def _interleave_perm():
    o = lax.broadcasted_iota(jnp.int32, (HW, HW), 1)
    si = lax.broadcasted_iota(jnp.int32, (HW, HW), 0)
    p = o % 32
    j = o - p
    want = j + jnp.where(p % 2 == 0, p // 2, 16 + p // 2)
    return (si == want).astype(jnp.float32)


def _project_tail(i, feat_list, ws_ref, wd_ref, ats_ref, atd_ref,
                  aug_ref, xs_ref, as_ref, ad_ref, ms_ref, md_ref):
    xs = sum(jnp.dot(f, ws_ref[pl.ds(o, f.shape[1]), :],
                     preferred_element_type=jnp.float32)
             for f, o in feat_list)
    xd = sum(jnp.dot(f, wd_ref[pl.ds(o, f.shape[1]), :],
                     preferred_element_type=jnp.float32)
             for f, o in feat_list)
    avs = jnp.sum(xs * ats_ref[...], axis=1, keepdims=True)
    avd = jnp.sum(xd * atd_ref[...], axis=1, keepdims=True)
    perm = _interleave_perm()
    xs_ref[...] = xs
    aug_ref[0] = jnp.dot(xs[:, :HW], perm,
                         preferred_element_type=jnp.float32).astype(jnp.bfloat16)
    aug_ref[1] = jnp.dot(xs[:, HW:], perm,
                         preferred_element_type=jnp.float32).astype(jnp.bfloat16)
    as_ref[...] = avs
    ad_ref[...] = avd

    @pl.when(i == 0)
    def _():
        ms_ref[...] = jnp.full((1, 1), -jnp.inf, jnp.float32)
        md_ref[...] = jnp.full((1, 1), -jnp.inf, jnp.float32)

    ms_ref[...] = jnp.maximum(ms_ref[...], jnp.broadcast_to(jnp.max(avs), (1, 1)))
    md_ref[...] = jnp.maximum(md_ref[...], jnp.broadcast_to(jnp.max(avd), (1, 1)))


def _normalize(part_ref, xsp_ref, asp_ref, adp_ref, msp_ref, mdp_ref,
               ba_ref, bb_ref):
    p0 = part_ref[0]
    p1 = part_ref[1]
    sel = jnp.where(
        lax.broadcasted_iota(jnp.int32, (SCW, 1), 0) == DEN, 1.0, 0.0
    ).astype(jnp.float32)
    den = jnp.dot(p0, sel, preferred_element_type=jnp.float32)
    m11 = jnp.maximum(msp_ref[...] + mdp_ref[...], 0.0)
    z = asp_ref[...] + adp_ref[...]
    alpha = jnp.where(z > 0, z, NEG * z)
    wself = jnp.exp(alpha - m11)
    xsp = xsp_ref[...]
    numa = p0[:, :HW] + wself * xsp[:, :HW]
    numb = p1[:, :HW] + wself * xsp[:, HW:]
    dent = den + wself + 1e-16
    ha = jnp.maximum(numa / dent + ba_ref[...], 0.0)
    hb = jnp.maximum(numb / dent + bb_ref[...], 0.0)
    return ha, hb


_NORM_SPECS = [
    pl.BlockSpec((NC, GB, SCW), lambda i: (0, i, 0)),
    pl.BlockSpec((GB, D), lambda i: (i, 0)),
    pl.BlockSpec((GB, 1), lambda i: (i, 0)),
    pl.BlockSpec((GB, 1), lambda i: (i, 0)),
    pl.BlockSpec((1, 1), lambda i: (0, 0)),
    pl.BlockSpec((1, 1), lambda i: (0, 0)),
    pl.BlockSpec((1, HW), lambda i: (0, 0)),
    pl.BlockSpec((1, HW), lambda i: (0, 0)),
]

_PROJ_IN_SPECS = [
    pl.BlockSpec((D, HID), lambda i: (0, 0)),
    pl.BlockSpec((D, HID), lambda i: (0, 0)),
    pl.BlockSpec((1, HID), lambda i: (0, 0)),
    pl.BlockSpec((1, HID), lambda i: (0, 0)),
]

_PROJ_OUT_SPECS = [
    pl.BlockSpec((NC, GB, HW), lambda i: (0, i, 0)),
    pl.BlockSpec((GB, D), lambda i: (i, 0)),
    pl.BlockSpec((GB, 1), lambda i: (i, 0)),
    pl.BlockSpec((GB, 1), lambda i: (i, 0)),
    pl.BlockSpec((1, 1), lambda i: (0, 0)),
    pl.BlockSpec((1, 1), lambda i: (0, 0)),
]

_PROJ_OUT_SHAPE = [
    jax.ShapeDtypeStruct((NC, NP, HW), jnp.bfloat16),
    jax.ShapeDtypeStruct((NP, D), jnp.float32),
    jax.ShapeDtypeStruct((NP, 1), jnp.float32),
    jax.ShapeDtypeStruct((NP, 1), jnp.float32),
    jax.ShapeDtypeStruct((1, 1), jnp.float32),
    jax.ShapeDtypeStruct((1, 1), jnp.float32),
]


def _project(feat, ws, wd, ats, atd):
    def body(f_ref, ws_ref, wd_ref, ats_ref, atd_ref,
             aug_ref, xs_ref, as_ref, ad_ref, ms_ref, md_ref):
        i = pl.program_id(0)
        _project_tail(i, [(f_ref[...], 0)], ws_ref, wd_ref, ats_ref, atd_ref,
                      aug_ref, xs_ref, as_ref, ad_ref, ms_ref, md_ref)

    return pl.pallas_call(
        body,
        grid=(GRID,),
        in_specs=[pl.BlockSpec((GB, D), lambda i: (i, 0))] + _PROJ_IN_SPECS,
        out_specs=_PROJ_OUT_SPECS,
        out_shape=_PROJ_OUT_SHAPE,
    )(feat, ws, wd, ats, atd)


def _norm_project(part, xsp, asp, adp, msp, mdp, ba, bb, ws, wd, ats, atd):
    def body(part_ref, xsp_ref, asp_ref, adp_ref, msp_ref, mdp_ref,
             ba_ref, bb_ref, ws_ref, wd_ref, ats_ref, atd_ref,
             aug_ref, xs_ref, as_ref, ad_ref, ms_ref, md_ref):
        i = pl.program_id(0)
        ha, hb = _normalize(part_ref, xsp_ref, asp_ref, adp_ref, msp_ref,
                            mdp_ref, ba_ref, bb_ref)
        _project_tail(i, [(ha, 0), (hb, HW)], ws_ref, wd_ref, ats_ref,
                      atd_ref, aug_ref, xs_ref, as_ref, ad_ref, ms_ref, md_ref)

    return pl.pallas_call(
        body,
        grid=(GRID,),
        in_specs=_NORM_SPECS + _PROJ_IN_SPECS,
        out_specs=_PROJ_OUT_SPECS,
        out_shape=_PROJ_OUT_SHAPE,
    )(part, xsp, asp, adp, msp, mdp, ba, bb, ws, wd, ats, atd)


def _head(part, xsp, asp, adp, msp, mdp, ba, bb,
          wf1, bf1, gamma, beta, wf2, bf2):
    bn_scale = float(1.0 / (1.0 + 1e-5) ** 0.5)

    def body(part_ref, xsp_ref, asp_ref, adp_ref, msp_ref, mdp_ref,
             ba_ref, bb_ref, wf1_ref, bf1_ref, g_ref, be_ref, wf2_ref,
             bf2_ref, y_ref):
        ha, hb = _normalize(part_ref, xsp_ref, asp_ref, adp_ref, msp_ref,
                            mdp_ref, ba_ref, bb_ref)
        f = (jnp.dot(ha, wf1_ref[pl.ds(0, HW), :],
                     preferred_element_type=jnp.float32)
             + jnp.dot(hb, wf1_ref[pl.ds(HW, HW), :],
                       preferred_element_type=jnp.float32))
        f = f + bf1_ref[...]
        f = g_ref[...] * (f * bn_scale) + be_ref[...]
        f = jnp.maximum(f, 0.0)
        g = jnp.dot(f, wf2_ref[...], preferred_element_type=jnp.float32)
        g = g + bf2_ref[...]
        mx = jnp.max(g, axis=1, keepdims=True)
        eg = jnp.exp(g - mx)
        y_ref[...] = g - mx - jnp.log(jnp.sum(eg, axis=1, keepdims=True))

    return pl.pallas_call(
        body,
        grid=(GRID,),
        in_specs=_NORM_SPECS + [
            pl.BlockSpec((HID, OUT), lambda i: (0, 0)),
            pl.BlockSpec((1, OUT), lambda i: (0, 0)),
            pl.BlockSpec((1, OUT), lambda i: (0, 0)),
            pl.BlockSpec((1, OUT), lambda i: (0, 0)),
            pl.BlockSpec((OUT, OUT), lambda i: (0, 0)),
            pl.BlockSpec((1, OUT), lambda i: (0, 0)),
        ],
        out_specs=pl.BlockSpec((GB, OUT), lambda i: (i, 0)),
        out_shape=jax.ShapeDtypeStruct((NP, OUT), jnp.float32),
    )(part, xsp, asp, adp, msp, mdp, ba, bb,
      wf1, bf1, gamma, beta, wf2, bf2)


def _edge_pass(src2, dst2, asrc, adst, mvec, xstk):
    mesh = plsc.VectorSubcoreMesh(core_axis_name="c", subcore_axis_name="s")

    @functools.partial(
        pl.kernel,
        out_type=jax.ShapeDtypeStruct((NC, NP, SCW), jnp.float32),
        mesh=mesh,
        compiler_params=pltpu.CompilerParams(
            needs_layout_passes=False, use_tc_tiling_on_sc=False),
        scratch_types=[
            pltpu.VMEM((NP,), jnp.float32),
            pltpu.VMEM((NP,), jnp.float32),
            pltpu.VMEM((L,), jnp.float32),
            pltpu.VMEM((CH,), jnp.int32),
            pltpu.VMEM((CH,), jnp.int32),
            pltpu.VMEM((CH,), jnp.int32),
            pltpu.VMEM((CH,), jnp.int32),
            pltpu.VMEM((CH,), jnp.float32),
            pltpu.VMEM((CH,), jnp.float32),
            pltpu.VMEM((CH, HW), jnp.bfloat16),
            pltpu.VMEM((CH, HW), jnp.bfloat16),
            pltpu.VMEM((CH, SCW), jnp.float32),
            pltpu.VMEM((CH, SCW), jnp.float32),
            pltpu.VMEM((ZR, SCW), jnp.float32),
            pltpu.VMEM_SHARED((NP, SCW), jnp.float32),
            pltpu.SemaphoreType.DMA,
            pltpu.SemaphoreType.DMA,
            pltpu.SemaphoreType.DMA,
            pltpu.SemaphoreType.DMA,
            pltpu.SemaphoreType.DMA,
            pltpu.SemaphoreType.DMA,
        ],
    )
    def k(src_h, dst_h, as_h, ad_h, m_h, xs_h, out_h,
          as_v, ad_v, m_v, si0, si1, di0, di1, w0, w1, gin0, gin1,
          rows0, rows1, z_v, acc_sh, isem0, isem1, gsem0, gsem1,
          ssem0, ssem1):
        c = lax.axis_index("c")
        s = lax.axis_index("s")

        pltpu.sync_copy(as_h, as_v)
        pltpu.sync_copy(ad_h, ad_v)
        pltpu.sync_copy(m_h, m_v)

        m16 = m_v[...]
        coff = jnp.full((L,), c * NP, jnp.int32)
        zeros16 = jnp.zeros((L,), jnp.float32)

        def zfill(r, carry):
            for j in range(SCW // L):
                z_v[r, pl.ds(j * L, L)] = zeros16
            return carry

        lax.fori_loop(0, ZR, zfill, 0)

        def bfill(r, carry):
            for j in range(SCW // L):
                rows0[r, pl.ds(j * L, L)] = zeros16
                rows1[r, pl.ds(j * L, L)] = zeros16
            return carry

        lax.fori_loop(0, CH, bfill, 0)

        row0 = s * RPT

        def zcopy(kk, carry):
            pltpu.sync_copy(z_v, acc_sh.at[pl.ds(row0 + kk * ZR, ZR)])
            return carry

        lax.fori_loop(0, RPT // ZR, zcopy, 0)
        plsc.subcore_barrier()

        sis = (si0, si1)
        dis = (di0, di1)
        ws = (w0, w1)
        gins = (gin0, gin1)
        bufs = (rows0, rows1)
        isems = (isem0, isem1)
        gsems = (gsem0, gsem1)
        ssems = (ssem0, ssem1)
        ebase = s * EPT
        lane = lax.iota(jnp.int32, L)

        def fire_idx(kk, b):
            base = ebase + kk * CH
            pltpu.async_copy(src_h.at[pl.ds(base, CH)], sis[b], isems[b])
            pltpu.async_copy(dst_h.at[pl.ds(base, CH)], dis[b], isems[b])

        def wait_idx(kk, b):
            base = ebase + kk * CH
            pltpu.make_async_copy(
                src_h.at[pl.ds(base, CH)], sis[b], isems[b]).wait()
            pltpu.make_async_copy(
                dst_h.at[pl.ds(base, CH)], dis[b], isems[b]).wait()

        fire_idx(0, 0)

        @pl.loop(0, NCHUNK, step=2)
        def _(kk0):
            for b in range(2):
                kk = kk0 + b
                nb = 1 - b
                si_b, di_b, w_b = sis[b], dis[b], ws[b]
                gin_b, rows_b = gins[b], bufs[b]

                wait_idx(kk, b)

                def wbody(i, carry2):
                    sl = pl.ds(i * L, L)
                    si = si_b[sl]
                    di = di_b[sl]
                    av = plsc.load_gather(as_v, [si])
                    bv = plsc.load_gather(ad_v, [di])
                    zz = av + bv
                    al = jnp.where(zz > 0, zz, NEG * zz)
                    w_b[sl] = jnp.exp(al - m16)
                    si_b[sl] = si + coff
                    return carry2

                lax.fori_loop(0, CH // L, wbody, 0)

                pltpu.async_copy(xs_h.at[si_b], gin_b, gsems[b])

                nxt = kk + 1

                @pl.when(nxt < NCHUNK)
                def _():
                    @pl.when(kk >= 1)
                    def _():
                        pltpu.make_async_copy(
                            bufs[nb], acc_sh.at[dis[nb]], ssems[nb]).wait()

                    fire_idx(nxt, nb)

                pltpu.make_async_copy(xs_h.at[si_b], gin_b, gsems[b]).wait()

                def mbody(r, carry2):
                    wb = plsc.load_gather(w_b, [jnp.full((L,), r, jnp.int32)])
                    for j in range(HW // (2 * L)):
                        v = gin_b[r, pl.ds(j * 2 * L, 2 * L)]
                        a, bb2 = plsc.unpack(
                            v, format=plsc.PackFormat.INTERLEAVED,
                            preferred_element_type=jnp.float32)
                        rows_b[r, pl.ds(j * 2 * L, L)] = a * wb
                        rows_b[r, pl.ds(j * 2 * L + L, L)] = bb2 * wb
                    return carry2

                lax.fori_loop(0, CH, mbody, 0)

                def dbody(i, carry2):
                    sl = pl.ds(i * L, L)
                    plsc.store_scatter(
                        rows_b, [lane + i * L, jnp.full((L,), DEN, jnp.int32)],
                        w_b[sl])
                    return carry2

                lax.fori_loop(0, CH // L, dbody, 0)

                pltpu.async_copy(rows_b, acc_sh.at[di_b], ssems[b], add=True)

        lastb = (NCHUNK - 1) % 2
        pltpu.make_async_copy(
            bufs[lastb], acc_sh.at[dis[lastb]], ssems[lastb]).wait()
        plsc.subcore_barrier()

        def wback(kk, carry):
            pltpu.sync_copy(acc_sh.at[pl.ds(row0 + kk * ZR, ZR)],
                            out_h.at[c, pl.ds(row0 + kk * ZR, ZR)])
            return carry

        lax.fori_loop(0, RPT // ZR, wback, 0)

    return k(src2, dst2, asrc, adst, mvec, xstk)


def kernel(x, edge_index, W1_src, W1_dst, att1_src, att1_dst, b1,
           W2_src, W2_dst, att2_src, att2_dst, b2,
           Wf1, bf1, gamma, beta, Wf2, bf2):
    xp = jnp.pad(x, ((0, NP - N), (0, 0)))
    src = jnp.pad(edge_index[0].astype(jnp.int32), (0, EP - E),
                  constant_values=N)
    dst = jnp.pad(edge_index[1].astype(jnp.int32), (0, EP - E),
                  constant_values=N)

    aug1, xs1, as1, ad1, ms1, md1 = _project(
        xp, W1_src, W1_dst, att1_src.reshape(1, -1), att1_dst.reshape(1, -1))
    m1 = jnp.maximum(ms1[0, 0] + md1[0, 0], 0.0)
    part1 = _edge_pass(src, dst, as1.reshape(NP), ad1.reshape(NP),
                       jnp.full((L,), 1.0, jnp.float32) * m1,
                       aug1.reshape(NC * NP, HW))

    b1a, b1b = b1[:HW].reshape(1, -1), b1[HW:].reshape(1, -1)
    aug2, xs2, as2, ad2, ms2, md2 = _norm_project(
        part1, xs1, as1, ad1, ms1, md1, b1a, b1b,
        W2_src, W2_dst, att2_src.reshape(1, -1), att2_dst.reshape(1, -1))
    m2 = jnp.maximum(ms2[0, 0] + md2[0, 0], 0.0)
    part2 = _edge_pass(src, dst, as2.reshape(NP), ad2.reshape(NP),
                       jnp.full((L,), 1.0, jnp.float32) * m2,
                       aug2.reshape(NC * NP, HW))

    b2a, b2b = b2[:HW].reshape(1, -1), b2[HW:].reshape(1, -1)
    y = _head(part2, xs2, as2, ad2, ms2, md2, b2a, b2b,
              Wf1, bf1.reshape(1, -1), gamma.reshape(1, -1),
              beta.reshape(1, -1), Wf2, bf2.reshape(1, -1))
    return y[:N]


# speedup vs baseline: 17.6368x; 1.0001x over previous
import functools

import jax
import jax.numpy as jnp
from jax import lax
from jax.experimental import pallas as pl
from jax.experimental.pallas import tpu as pltpu
from jax.experimental.pallas import tpu_sc as plsc

N = 10000
D = 128
E = 320000
HID = 128
OUT = 64
NEG = 0.2

NP = 10240
DW = 16
GB = 1024
GRID = NP // GB

NC, NS, L = 2, 16, 16
NT = NC * NS
CH = 64
NCHUNK = 160
EPT = CH * NCHUNK
EP = NT * EPT
RPT = NP // NS


def _project_tail(i, feat_list, ws_ref, wd_ref, ats_ref, atd_ref,
                  xs_ref, pk_ref, as_ref, ad_ref, ms_ref, md_ref):
    xs = sum(jnp.dot(f, ws_ref[pl.ds(o, f.shape[1]), :],
                     preferred_element_type=jnp.float32)
             for f, o in feat_list)
    xd = sum(jnp.dot(f, wd_ref[pl.ds(o, f.shape[1]), :],
                     preferred_element_type=jnp.float32)
             for f, o in feat_list)
    avs = jnp.sum(xs * ats_ref[...], axis=1, keepdims=True)
    avd = jnp.sum(xd * atd_ref[...], axis=1, keepdims=True)
    xs_ref[...] = xs
    bs = lax.bitcast_convert_type(avs, jnp.uint32)
    bd = lax.bitcast_convert_type(avd, jnp.uint32)
    packed = (bd & jnp.uint32(0xFFFF0000)) | (bs >> 16)
    pk_ref[...] = lax.bitcast_convert_type(packed, jnp.int32)
    as_ref[...] = avs
    ad_ref[...] = avd

    @pl.when(i == 0)
    def _():
        ms_ref[...] = jnp.full((1, 1), -jnp.inf, jnp.float32)
        md_ref[...] = jnp.full((1, 1), -jnp.inf, jnp.float32)

    ms_ref[...] = jnp.maximum(ms_ref[...], jnp.broadcast_to(jnp.max(avs), (1, 1)))
    md_ref[...] = jnp.maximum(md_ref[...], jnp.broadcast_to(jnp.max(avd), (1, 1)))


def _normalize(pf_ref, pd_ref, xsp_ref, asp_ref, adp_ref, msp_ref, mdp_ref,
               b_ref):
    num = pf_ref[0] + pf_ref[1]
    dsum = pd_ref[0] + pd_ref[1]
    den = dsum[:, :1]
    m11 = jnp.maximum(msp_ref[...] + mdp_ref[...], 0.0)
    z = asp_ref[...] + adp_ref[...]
    alpha = jnp.where(z > 0, z, NEG * z)
    wself = jnp.exp(alpha - m11)
    numt = num + wself * xsp_ref[...]
    dent = den + wself + 1e-16
    return jnp.maximum(numt / dent + b_ref[...], 0.0)


_NORM_SPECS = [
    pl.BlockSpec((NC, GB, D), lambda i: (0, i, 0)),
    pl.BlockSpec((NC, GB, DW), lambda i: (0, i, 0)),
    pl.BlockSpec((GB, D), lambda i: (i, 0)),
    pl.BlockSpec((GB, 1), lambda i: (i, 0)),
    pl.BlockSpec((GB, 1), lambda i: (i, 0)),
    pl.BlockSpec((1, 1), lambda i: (0, 0)),
    pl.BlockSpec((1, 1), lambda i: (0, 0)),
    pl.BlockSpec((1, HID), lambda i: (0, 0)),
]

_PROJ_IN_SPECS = [
    pl.BlockSpec((D, HID), lambda i: (0, 0)),
    pl.BlockSpec((D, HID), lambda i: (0, 0)),
    pl.BlockSpec((1, HID), lambda i: (0, 0)),
    pl.BlockSpec((1, HID), lambda i: (0, 0)),
]

_PROJ_OUT_SPECS = [
    pl.BlockSpec((GB, D), lambda i: (i, 0)),
    pl.BlockSpec((GB, 1), lambda i: (i, 0)),
    pl.BlockSpec((GB, 1), lambda i: (i, 0)),
    pl.BlockSpec((GB, 1), lambda i: (i, 0)),
    pl.BlockSpec((1, 1), lambda i: (0, 0)),
    pl.BlockSpec((1, 1), lambda i: (0, 0)),
]

_PROJ_OUT_SHAPE = [
    jax.ShapeDtypeStruct((NP, D), jnp.float32),
    jax.ShapeDtypeStruct((NP, 1), jnp.int32),
    jax.ShapeDtypeStruct((NP, 1), jnp.float32),
    jax.ShapeDtypeStruct((NP, 1), jnp.float32),
    jax.ShapeDtypeStruct((1, 1), jnp.float32),
    jax.ShapeDtypeStruct((1, 1), jnp.float32),
]


def _project(feat, ws, wd, ats, atd):
    def body(f_ref, ws_ref, wd_ref, ats_ref, atd_ref,
             xs_ref, pk_ref, as_ref, ad_ref, ms_ref, md_ref):
        i = pl.program_id(0)
        _project_tail(i, [(f_ref[...], 0)], ws_ref, wd_ref, ats_ref, atd_ref,
                      xs_ref, pk_ref, as_ref, ad_ref, ms_ref, md_ref)

    return pl.pallas_call(
        body,
        grid=(GRID,),
        in_specs=[pl.BlockSpec((GB, D), lambda i: (i, 0))] + _PROJ_IN_SPECS,
        out_specs=_PROJ_OUT_SPECS,
        out_shape=_PROJ_OUT_SHAPE,
    )(feat, ws, wd, ats, atd)


def _norm_project(pf, pd, xsp, asp, adp, msp, mdp, b, ws, wd, ats, atd):
    def body(pf_ref, pd_ref, xsp_ref, asp_ref, adp_ref, msp_ref, mdp_ref,
             b_ref, ws_ref, wd_ref, ats_ref, atd_ref,
             xs_ref, pk_ref, as_ref, ad_ref, ms_ref, md_ref):
        i = pl.program_id(0)
        h = _normalize(pf_ref, pd_ref, xsp_ref, asp_ref, adp_ref, msp_ref,
                       mdp_ref, b_ref)
        _project_tail(i, [(h, 0)], ws_ref, wd_ref, ats_ref, atd_ref,
                      xs_ref, pk_ref, as_ref, ad_ref, ms_ref, md_ref)

    return pl.pallas_call(
        body,
        grid=(GRID,),
        in_specs=_NORM_SPECS + _PROJ_IN_SPECS,
        out_specs=_PROJ_OUT_SPECS,
        out_shape=_PROJ_OUT_SHAPE,
    )(pf, pd, xsp, asp, adp, msp, mdp, b, ws, wd, ats, atd)


def _head(pf, pd, xsp, asp, adp, msp, mdp, b,
          wf1, bf1, gamma, beta, wf2, bf2):
    bn_scale = float(1.0 / (1.0 + 1e-5) ** 0.5)

    def body(pf_ref, pd_ref, xsp_ref, asp_ref, adp_ref, msp_ref, mdp_ref,
             b_ref, wf1_ref, bf1_ref, g_ref, be_ref, wf2_ref, bf2_ref,
             y_ref):
        h = _normalize(pf_ref, pd_ref, xsp_ref, asp_ref, adp_ref, msp_ref,
                       mdp_ref, b_ref)
        f = jnp.dot(h, wf1_ref[...], preferred_element_type=jnp.float32)
        f = f + bf1_ref[...]
        f = g_ref[...] * (f * bn_scale) + be_ref[...]
        f = jnp.maximum(f, 0.0)
        g = jnp.dot(f, wf2_ref[...], preferred_element_type=jnp.float32)
        g = g + bf2_ref[...]
        mx = jnp.max(g, axis=1, keepdims=True)
        eg = jnp.exp(g - mx)
        y_ref[...] = g - mx - jnp.log(jnp.sum(eg, axis=1, keepdims=True))

    return pl.pallas_call(
        body,
        grid=(GRID,),
        in_specs=_NORM_SPECS + [
            pl.BlockSpec((HID, OUT), lambda i: (0, 0)),
            pl.BlockSpec((1, OUT), lambda i: (0, 0)),
            pl.BlockSpec((1, OUT), lambda i: (0, 0)),
            pl.BlockSpec((1, OUT), lambda i: (0, 0)),
            pl.BlockSpec((OUT, OUT), lambda i: (0, 0)),
            pl.BlockSpec((1, OUT), lambda i: (0, 0)),
        ],
        out_specs=pl.BlockSpec((GB, OUT), lambda i: (i, 0)),
        out_shape=jax.ShapeDtypeStruct((NP, OUT), jnp.float32),
    )(pf, pd, xsp, asp, adp, msp, mdp, b,
      wf1, bf1, gamma, beta, wf2, bf2)


def _edge_pass(src2, dst2, packed, mvec, xsf):
    mesh = plsc.VectorSubcoreMesh(core_axis_name="c", subcore_axis_name="s")

    @functools.partial(
        pl.kernel,
        out_type=[jax.ShapeDtypeStruct((NC, NP, D), jnp.float32),
                  jax.ShapeDtypeStruct((NC, NP, DW), jnp.float32)],
        mesh=mesh,
        compiler_params=pltpu.CompilerParams(
            needs_layout_passes=False, use_tc_tiling_on_sc=False),
        scratch_types=[
            pltpu.VMEM((NP,), jnp.int32),
            pltpu.VMEM((L,), jnp.float32),
            pltpu.VMEM((CH,), jnp.int32),
            pltpu.VMEM((CH,), jnp.int32),
            pltpu.VMEM((CH,), jnp.int32),
            pltpu.VMEM((CH,), jnp.int32),
            pltpu.VMEM((CH,), jnp.float32),
            pltpu.VMEM((CH,), jnp.float32),
            pltpu.VMEM((CH, D), jnp.float32),
            pltpu.VMEM((CH, D), jnp.float32),
            pltpu.VMEM((CH, DW), jnp.float32),
            pltpu.VMEM((CH, DW), jnp.float32),
            pltpu.VMEM_SHARED((NP, D), jnp.float32),
            pltpu.VMEM_SHARED((NP, DW), jnp.float32),
            pltpu.SemaphoreType.DMA,
            pltpu.SemaphoreType.DMA,
            pltpu.SemaphoreType.DMA,
            pltpu.SemaphoreType.DMA,
            pltpu.SemaphoreType.DMA,
            pltpu.SemaphoreType.DMA,
            pltpu.SemaphoreType.DMA,
            pltpu.SemaphoreType.DMA,
        ],
    )
    def k(src_h, dst_h, pk_h, m_h, xs_h, outf_h, outd_h,
          pk_v, m_v, si0, si1, di0, di1, w0, w1, rows0, rows1, wr0, wr1,
          accf_sh, accd_sh, isem0, isem1, gsem0, gsem1, ssem0, ssem1,
          dsem0, dsem1):
        c = lax.axis_index("c")
        s = lax.axis_index("s")

        pltpu.sync_copy(pk_h, pk_v)
        pltpu.sync_copy(m_h, m_v)

        m16 = m_v[...]
        zeros16 = jnp.zeros((L,), jnp.float32)
        himask = jnp.full((L,), -65536, jnp.int32)

        def bfill(r, carry):
            for j in range(D // L):
                rows0[r, pl.ds(j * L, L)] = zeros16
                rows1[r, pl.ds(j * L, L)] = zeros16
            rows_wslice = pl.ds(0, L)
            wr0[r, rows_wslice] = zeros16
            wr1[r, rows_wslice] = zeros16
            return carry

        lax.fori_loop(0, CH, bfill, 0)

        row0 = s * RPT

        def zcopy(kk, carry):
            pltpu.sync_copy(rows0, accf_sh.at[pl.ds(row0 + kk * CH, CH)])
            pltpu.sync_copy(wr0, accd_sh.at[pl.ds(row0 + kk * CH, CH)])
            return carry

        lax.fori_loop(0, RPT // CH, zcopy, 0)
        plsc.subcore_barrier()

        sis = (si0, si1)
        dis = (di0, di1)
        ws = (w0, w1)
        bufs = (rows0, rows1)
        wrs = (wr0, wr1)
        isems = (isem0, isem1)
        gsems = (gsem0, gsem1)
        ssems = (ssem0, ssem1)
        dsems = (dsem0, dsem1)
        ebase = (c * NS + s) * EPT
        lane = lax.iota(jnp.int32, L)

        def fire_idx(kk, b):
            base = ebase + kk * CH
            pltpu.async_copy(src_h.at[pl.ds(base, CH)], sis[b], isems[b])
            pltpu.async_copy(dst_h.at[pl.ds(base, CH)], dis[b], isems[b])

        def wait_idx(kk, b):
            base = ebase + kk * CH
            pltpu.make_async_copy(
                src_h.at[pl.ds(base, CH)], sis[b], isems[b]).wait()
            pltpu.make_async_copy(
                dst_h.at[pl.ds(base, CH)], dis[b], isems[b]).wait()

        fire_idx(0, 0)

        @pl.loop(0, NCHUNK, step=2)
        def _(kk0):
            for b in range(2):
                kk = kk0 + b
                nb = 1 - b
                si_b, di_b, w_b = sis[b], dis[b], ws[b]
                rows_b, wr_b = bufs[b], wrs[b]

                wait_idx(kk, b)

                def wbody(i, carry2):
                    sl = pl.ds(i * L, L)
                    ps = plsc.load_gather(pk_v, [si_b[sl]])
                    pd = plsc.load_gather(pk_v, [di_b[sl]])
                    a_s = plsc.bitcast(lax.shift_left(ps, 16), jnp.float32)
                    a_d = plsc.bitcast(pd & himask, jnp.float32)
                    zz = a_s + a_d
                    al = jnp.where(zz > 0, zz, NEG * zz)
                    w = jnp.exp(al - m16)
                    w_b[sl] = w
                    plsc.store_scatter(
                        wr_b, [lane + i * L, jnp.full((L,), 0, jnp.int32)], w)
                    return carry2

                pltpu.async_copy(xs_h.at[si_b], rows_b, gsems[b])
                lax.fori_loop(0, CH // L, wbody, 0)

                nxt = kk + 1

                @pl.when(nxt < NCHUNK)
                def _():
                    @pl.when(kk >= 1)
                    def _():
                        pltpu.make_async_copy(
                            bufs[nb], accf_sh.at[dis[nb]], ssems[nb]).wait()
                        pltpu.make_async_copy(
                            wrs[nb], accd_sh.at[dis[nb]], dsems[nb]).wait()

                    fire_idx(nxt, nb)

                pltpu.make_async_copy(xs_h.at[si_b], rows_b, gsems[b]).wait()

                def mbody(r, carry2):
                    wb = plsc.load_gather(w_b, [jnp.full((L,), r, jnp.int32)])
                    for j in range(D // L):
                        rows_b[r, pl.ds(j * L, L)] = (
                            rows_b[r, pl.ds(j * L, L)] * wb)
                    return carry2

                lax.fori_loop(0, CH, mbody, 0)

                pltpu.async_copy(rows_b, accf_sh.at[di_b], ssems[b], add=True)
                pltpu.async_copy(wr_b, accd_sh.at[di_b], dsems[b], add=True)

        for b in ((NCHUNK - 2) % 2, (NCHUNK - 1) % 2):
            pltpu.make_async_copy(
                bufs[b], accf_sh.at[dis[b]], ssems[b]).wait()
            pltpu.make_async_copy(
                wrs[b], accd_sh.at[dis[b]], dsems[b]).wait()
        plsc.subcore_barrier()

        def wback(kk, carry):
            sl = pl.ds(row0 + kk * CH, CH)
            pltpu.sync_copy(accf_sh.at[sl], outf_h.at[c, sl])
            pltpu.sync_copy(accd_sh.at[sl], outd_h.at[c, sl])
            return carry

        lax.fori_loop(0, RPT // CH, wback, 0)

    return k(src2, dst2, packed, mvec, xsf)


def kernel(x, edge_index, W1_src, W1_dst, att1_src, att1_dst, b1,
           W2_src, W2_dst, att2_src, att2_dst, b2,
           Wf1, bf1, gamma, beta, Wf2, bf2):
    xp = jnp.pad(x, ((0, NP - N), (0, 0)))
    src = jnp.pad(edge_index[0].astype(jnp.int32), (0, EP - E),
                  constant_values=N)
    dst = jnp.pad(edge_index[1].astype(jnp.int32), (0, EP - E),
                  constant_values=N)

    xs1, pk1, as1, ad1, ms1, md1 = _project(
        xp, W1_src, W1_dst, att1_src.reshape(1, -1), att1_dst.reshape(1, -1))
    m1 = jnp.maximum(ms1[0, 0] + md1[0, 0], 0.0)
    pf1, pd1 = _edge_pass(src, dst, pk1.reshape(NP),
                          jnp.full((L,), 1.0, jnp.float32) * m1, xs1)

    xs2, pk2, as2, ad2, ms2, md2 = _norm_project(
        pf1, pd1, xs1, as1, ad1, ms1, md1, b1.reshape(1, -1),
        W2_src, W2_dst, att2_src.reshape(1, -1), att2_dst.reshape(1, -1))
    m2 = jnp.maximum(ms2[0, 0] + md2[0, 0], 0.0)
    pf2, pd2 = _edge_pass(src, dst, pk2.reshape(NP),
                          jnp.full((L,), 1.0, jnp.float32) * m2, xs2)

    y = _head(pf2, pd2, xs2, as2, ad2, ms2, md2, b2.reshape(1, -1),
              Wf1, bf1.reshape(1, -1), gamma.reshape(1, -1),
              beta.reshape(1, -1), Wf2, bf2.reshape(1, -1))
    return y[:N]


# speedup vs baseline: 23.0626x; 1.3076x over previous
import functools

import jax
import jax.numpy as jnp
from jax import lax
from jax.experimental import pallas as pl
from jax.experimental.pallas import tpu as pltpu
from jax.experimental.pallas import tpu_sc as plsc

N = 10000
D = 128
E = 320000
HID = 128
OUT = 64
NEG = 0.2

NP = 10240
SCW = 80
WA = 80
WB = 48
DEN = 48
GB = 1024
GRID = NP // GB

NC, NS, L = 2, 16, 16
EPT = E // NS
CH = 80
NCHUNK = EPT // CH
RPT = NP // NS
ZR = 64


def _project_tail(i, feat_list, ws_ref, wd_ref, ats_ref, atd_ref,
                  aug_ref, as_ref, ad_ref, ms_ref, md_ref):
    xs = sum(jnp.dot(f, ws_ref[pl.ds(o, f.shape[1]), :],
                     preferred_element_type=jnp.float32)
             for f, o in feat_list)
    xd = sum(jnp.dot(f, wd_ref[pl.ds(o, f.shape[1]), :],
                     preferred_element_type=jnp.float32)
             for f, o in feat_list)
    avs = jnp.sum(xs * ats_ref[...], axis=1, keepdims=True)
    avd = jnp.sum(xd * atd_ref[...], axis=1, keepdims=True)
    ones = jnp.ones((GB, 1), jnp.float32)
    zpad = jnp.zeros((GB, SCW - WB - 1), jnp.float32)
    aug_ref[0] = xs[:, :WA]
    aug_ref[1] = jnp.concatenate([xs[:, WA:D], ones, zpad], axis=1)
    as_ref[...] = avs
    ad_ref[...] = avd

    @pl.when(i == 0)
    def _():
        ms_ref[...] = jnp.full((1, 1), -jnp.inf, jnp.float32)
        md_ref[...] = jnp.full((1, 1), -jnp.inf, jnp.float32)

    ms_ref[...] = jnp.maximum(ms_ref[...], jnp.broadcast_to(jnp.max(avs), (1, 1)))
    md_ref[...] = jnp.maximum(md_ref[...], jnp.broadcast_to(jnp.max(avd), (1, 1)))


def _normalize(part_ref, augp_ref, asp_ref, adp_ref, msp_ref, mdp_ref,
               ba_ref, bb_ref):
    p0 = part_ref[0]
    p1 = part_ref[1]
    sel = jnp.where(
        lax.broadcasted_iota(jnp.int32, (SCW, 1), 0) == DEN, 1.0, 0.0
    ).astype(jnp.float32)
    den = jnp.dot(p1, sel, preferred_element_type=jnp.float32)
    m11 = jnp.maximum(msp_ref[...] + mdp_ref[...], 0.0)
    z = asp_ref[...] + adp_ref[...]
    alpha = jnp.where(z > 0, z, NEG * z)
    wself = jnp.exp(alpha - m11)
    numa = p0 + wself * augp_ref[0]
    numb = p1[:, :WB] + wself * augp_ref[1][:, :WB]
    dent = den + wself + 1e-16
    ha = jnp.maximum(numa / dent + ba_ref[...], 0.0)
    hb = jnp.maximum(numb / dent + bb_ref[...], 0.0)
    return ha, hb


_NORM_SPECS = [
    pl.BlockSpec((NC, GB, SCW), lambda i: (0, i, 0)),
    pl.BlockSpec((NC, GB, SCW), lambda i: (0, i, 0)),
    pl.BlockSpec((GB, 1), lambda i: (i, 0)),
    pl.BlockSpec((GB, 1), lambda i: (i, 0)),
    pl.BlockSpec((1, 1), lambda i: (0, 0)),
    pl.BlockSpec((1, 1), lambda i: (0, 0)),
    pl.BlockSpec((1, WA), lambda i: (0, 0)),
    pl.BlockSpec((1, WB), lambda i: (0, 0)),
]

_PROJ_IN_SPECS = [
    pl.BlockSpec((D, HID), lambda i: (0, 0)),
    pl.BlockSpec((D, HID), lambda i: (0, 0)),
    pl.BlockSpec((1, HID), lambda i: (0, 0)),
    pl.BlockSpec((1, HID), lambda i: (0, 0)),
]

_PROJ_OUT_SPECS = [
    pl.BlockSpec((NC, GB, SCW), lambda i: (0, i, 0)),
    pl.BlockSpec((GB, 1), lambda i: (i, 0)),
    pl.BlockSpec((GB, 1), lambda i: (i, 0)),
    pl.BlockSpec((1, 1), lambda i: (0, 0)),
    pl.BlockSpec((1, 1), lambda i: (0, 0)),
]

_PROJ_OUT_SHAPE = [
    jax.ShapeDtypeStruct((NC, NP, SCW), jnp.float32),
    jax.ShapeDtypeStruct((NP, 1), jnp.float32),
    jax.ShapeDtypeStruct((NP, 1), jnp.float32),
    jax.ShapeDtypeStruct((1, 1), jnp.float32),
    jax.ShapeDtypeStruct((1, 1), jnp.float32),
]


def _project(feat, ws, wd, ats, atd):
    def body(f_ref, ws_ref, wd_ref, ats_ref, atd_ref,
             aug_ref, as_ref, ad_ref, ms_ref, md_ref):
        i = pl.program_id(0)
        _project_tail(i, [(f_ref[...], 0)], ws_ref, wd_ref, ats_ref, atd_ref,
                      aug_ref, as_ref, ad_ref, ms_ref, md_ref)

    return pl.pallas_call(
        body,
        grid=(GRID,),
        in_specs=[pl.BlockSpec((GB, D), lambda i: (i, 0))] + _PROJ_IN_SPECS,
        out_specs=_PROJ_OUT_SPECS,
        out_shape=_PROJ_OUT_SHAPE,
    )(feat, ws, wd, ats, atd)


def _norm_project(part, augp, asp, adp, msp, mdp, ba, bb, ws, wd, ats, atd):
    def body(part_ref, augp_ref, asp_ref, adp_ref, msp_ref, mdp_ref,
             ba_ref, bb_ref, ws_ref, wd_ref, ats_ref, atd_ref,
             aug_ref, as_ref, ad_ref, ms_ref, md_ref):
        i = pl.program_id(0)
        ha, hb = _normalize(part_ref, augp_ref, asp_ref, adp_ref, msp_ref,
                            mdp_ref, ba_ref, bb_ref)
        _project_tail(i, [(ha, 0), (hb, WA)], ws_ref, wd_ref, ats_ref,
                      atd_ref, aug_ref, as_ref, ad_ref, ms_ref, md_ref)

    return pl.pallas_call(
        body,
        grid=(GRID,),
        in_specs=_NORM_SPECS + _PROJ_IN_SPECS,
        out_specs=_PROJ_OUT_SPECS,
        out_shape=_PROJ_OUT_SHAPE,
    )(part, augp, asp, adp, msp, mdp, ba, bb, ws, wd, ats, atd)


def _head(part, augp, asp, adp, msp, mdp, ba, bb,
          wf1, bf1, gamma, beta, wf2, bf2):
    bn_scale = float(1.0 / (1.0 + 1e-5) ** 0.5)

    def body(part_ref, augp_ref, asp_ref, adp_ref, msp_ref, mdp_ref,
             ba_ref, bb_ref, wf1_ref, bf1_ref, g_ref, be_ref, wf2_ref,
             bf2_ref, y_ref):
        ha, hb = _normalize(part_ref, augp_ref, asp_ref, adp_ref, msp_ref,
                            mdp_ref, ba_ref, bb_ref)
        f = (jnp.dot(ha, wf1_ref[pl.ds(0, WA), :],
                     preferred_element_type=jnp.float32)
             + jnp.dot(hb, wf1_ref[pl.ds(WA, WB), :],
                       preferred_element_type=jnp.float32))
        f = f + bf1_ref[...]
        f = g_ref[...] * (f * bn_scale) + be_ref[...]
        f = jnp.maximum(f, 0.0)
        g = jnp.dot(f, wf2_ref[...], preferred_element_type=jnp.float32)
        g = g + bf2_ref[...]
        mx = jnp.max(g, axis=1, keepdims=True)
        eg = jnp.exp(g - mx)
        y_ref[...] = g - mx - jnp.log(jnp.sum(eg, axis=1, keepdims=True))

    return pl.pallas_call(
        body,
        grid=(GRID,),
        in_specs=_NORM_SPECS + [
            pl.BlockSpec((HID, OUT), lambda i: (0, 0)),
            pl.BlockSpec((1, OUT), lambda i: (0, 0)),
            pl.BlockSpec((1, OUT), lambda i: (0, 0)),
            pl.BlockSpec((1, OUT), lambda i: (0, 0)),
            pl.BlockSpec((OUT, OUT), lambda i: (0, 0)),
            pl.BlockSpec((1, OUT), lambda i: (0, 0)),
        ],
        out_specs=pl.BlockSpec((GB, OUT), lambda i: (i, 0)),
        out_shape=jax.ShapeDtypeStruct((NP, OUT), jnp.float32),
    )(part, augp, asp, adp, msp, mdp, ba, bb,
      wf1, bf1, gamma, beta, wf2, bf2)


def _edge_pass(src2, dst2, asrc, adst, mvec, xstk):
    mesh = plsc.VectorSubcoreMesh(core_axis_name="c", subcore_axis_name="s")

    @functools.partial(
        pl.kernel,
        out_type=jax.ShapeDtypeStruct((NC, NP, SCW), jnp.float32),
        mesh=mesh,
        compiler_params=pltpu.CompilerParams(
            needs_layout_passes=False, use_tc_tiling_on_sc=False),
        scratch_types=[
            pltpu.VMEM((NP,), jnp.float32),
            pltpu.VMEM((NP,), jnp.float32),
            pltpu.VMEM((L,), jnp.float32),
            pltpu.VMEM((CH,), jnp.int32),
            pltpu.VMEM((CH,), jnp.int32),
            pltpu.VMEM((CH,), jnp.int32),
            pltpu.VMEM((CH,), jnp.int32),
            pltpu.VMEM((CH,), jnp.float32),
            pltpu.VMEM((CH,), jnp.float32),
            pltpu.VMEM((CH, SCW), jnp.float32),
            pltpu.VMEM((CH, SCW), jnp.float32),
            pltpu.VMEM((ZR, SCW), jnp.float32),
            pltpu.VMEM_SHARED((NP, SCW), jnp.float32),
            pltpu.SemaphoreType.DMA,
            pltpu.SemaphoreType.DMA,
            pltpu.SemaphoreType.DMA,
            pltpu.SemaphoreType.DMA,
            pltpu.SemaphoreType.DMA,
            pltpu.SemaphoreType.DMA,
        ],
    )
    def k(src_h, dst_h, as_h, ad_h, m_h, xs_h, out_h,
          as_v, ad_v, m_v, si0, si1, di0, di1, w0, w1, rows0, rows1,
          z_v, acc_sh, isem0, isem1, gsem0, gsem1, ssem0, ssem1):
        c = lax.axis_index("c")
        s = lax.axis_index("s")

        pltpu.sync_copy(as_h, as_v)
        pltpu.sync_copy(ad_h, ad_v)
        pltpu.sync_copy(m_h, m_v)

        m16 = m_v[...]
        coff = jnp.full((L,), c * NP, jnp.int32)
        zeros16 = jnp.zeros((L,), jnp.float32)

        def zfill(r, carry):
            for j in range(SCW // L):
                z_v[r, pl.ds(j * L, L)] = zeros16
            return carry

        lax.fori_loop(0, ZR, zfill, 0)

        row0 = s * RPT

        def zcopy(kk, carry):
            pltpu.sync_copy(z_v, acc_sh.at[pl.ds(row0 + kk * ZR, ZR)])
            return carry

        lax.fori_loop(0, RPT // ZR, zcopy, 0)
        plsc.subcore_barrier()

        sis = (si0, si1)
        dis = (di0, di1)
        ws = (w0, w1)
        bufs = (rows0, rows1)
        isems = (isem0, isem1)
        gsems = (gsem0, gsem1)
        ssems = (ssem0, ssem1)
        ebase = s * EPT

        def fire_idx(kk, b):
            base = ebase + kk * CH
            pltpu.async_copy(src_h.at[pl.ds(base, CH)], sis[b], isems[b])
            pltpu.async_copy(dst_h.at[pl.ds(base, CH)], dis[b], isems[b])

        def wait_idx(kk, b):
            base = ebase + kk * CH
            pltpu.make_async_copy(
                src_h.at[pl.ds(base, CH)], sis[b], isems[b]).wait()
            pltpu.make_async_copy(
                dst_h.at[pl.ds(base, CH)], dis[b], isems[b]).wait()

        fire_idx(0, 0)

        @pl.loop(0, NCHUNK, step=2)
        def _(kk0):
            for b in range(2):
                kk = kk0 + b
                nb = 1 - b
                si_b, di_b, w_b, rows_b = sis[b], dis[b], ws[b], bufs[b]

                wait_idx(kk, b)

                def wbody(i, carry2):
                    sl = pl.ds(i * L, L)
                    si = si_b[sl]
                    di = di_b[sl]
                    av = plsc.load_gather(as_v, [si])
                    bv = plsc.load_gather(ad_v, [di])
                    zz = av + bv
                    al = jnp.where(zz > 0, zz, NEG * zz)
                    w_b[sl] = jnp.exp(al - m16)
                    si_b[sl] = si + coff
                    return carry2

                lax.fori_loop(0, CH // L, wbody, 0)

                pltpu.async_copy(xs_h.at[si_b], rows_b, gsems[b])

                nxt = kk + 1

                @pl.when(nxt < NCHUNK)
                def _():
                    @pl.when(kk >= 1)
                    def _():
                        pltpu.make_async_copy(
                            bufs[nb], acc_sh.at[dis[nb]], ssems[nb]).wait()

                    fire_idx(nxt, nb)

                pltpu.make_async_copy(xs_h.at[si_b], rows_b, gsems[b]).wait()

                def mbody(r, carry2):
                    wb = plsc.load_gather(w_b, [jnp.full((L,), r, jnp.int32)])
                    for j in range(SCW // L):
                        rows_b[r, pl.ds(j * L, L)] = (
                            rows_b[r, pl.ds(j * L, L)] * wb)
                    return carry2

                lax.fori_loop(0, CH, mbody, 0)

                pltpu.async_copy(rows_b, acc_sh.at[di_b], ssems[b], add=True)

        for b in ((NCHUNK - 2) % 2, (NCHUNK - 1) % 2):
            pltpu.make_async_copy(
                bufs[b], acc_sh.at[dis[b]], ssems[b]).wait()
        plsc.subcore_barrier()

        def wback(kk, carry):
            pltpu.sync_copy(acc_sh.at[pl.ds(row0 + kk * ZR, ZR)],
                            out_h.at[c, pl.ds(row0 + kk * ZR, ZR)])
            return carry

        lax.fori_loop(0, RPT // ZR, wback, 0)

    return k(src2, dst2, asrc, adst, mvec, xstk)


def kernel(x, edge_index, W1_src, W1_dst, att1_src, att1_dst, b1,
           W2_src, W2_dst, att2_src, att2_dst, b2,
           Wf1, bf1, gamma, beta, Wf2, bf2):
    xp = jnp.pad(x, ((0, NP - N), (0, 0)))
    src = edge_index[0].astype(jnp.int32)
    dst = edge_index[1].astype(jnp.int32)

    aug1, as1, ad1, ms1, md1 = _project(
        xp, W1_src, W1_dst, att1_src.reshape(1, -1), att1_dst.reshape(1, -1))
    m1 = jnp.maximum(ms1[0, 0] + md1[0, 0], 0.0)
    part1 = _edge_pass(src, dst, as1.reshape(NP), ad1.reshape(NP),
                       jnp.full((L,), 1.0, jnp.float32) * m1,
                       aug1.reshape(NC * NP, SCW))

    b1a, b1b = b1[:WA].reshape(1, -1), b1[WA:].reshape(1, -1)
    aug2, as2, ad2, ms2, md2 = _norm_project(
        part1, aug1, as1, ad1, ms1, md1, b1a, b1b,
        W2_src, W2_dst, att2_src.reshape(1, -1), att2_dst.reshape(1, -1))
    m2 = jnp.maximum(ms2[0, 0] + md2[0, 0], 0.0)
    part2 = _edge_pass(src, dst, as2.reshape(NP), ad2.reshape(NP),
                       jnp.full((L,), 1.0, jnp.float32) * m2,
                       aug2.reshape(NC * NP, SCW))

    b2a, b2b = b2[:WA].reshape(1, -1), b2[WA:].reshape(1, -1)
    y = _head(part2, aug2, as2, ad2, ms2, md2, b2a, b2b,
              Wf1, bf1.reshape(1, -1), gamma.reshape(1, -1),
              beta.reshape(1, -1), Wf2, bf2.reshape(1, -1))
    return y[:N]
